# baseline (speedup 1.0000x reference)
import functools

import jax
import jax.numpy as jnp
from jax import lax
from jax.experimental import pallas as pl
from jax.experimental.pallas import tpu as pltpu
from jax.experimental.pallas import tpu_sc as plsc

N_NODES = 100000
HIDDEN = 128
NUM_SEGMENTS = 1024

BLK = 1024
NBLK = (N_NODES + BLK - 1) // BLK
N_PAD = NBLK * BLK
W = 32

CHUNK = 128
CPB = BLK // CHUNK
N_CHUNKS = NBLK * CPB
LAST_FULL_CHUNK = N_NODES // CHUNK - 1
TAIL_CHUNK = 781
TAIL_ROWS = N_NODES - TAIL_CHUNK * CHUNK

NUM_WORKERS = 32
NS = 16
SEG_PER_SUB = NUM_SEGMENTS // NS
ROUNDS = (NBLK + NUM_WORKERS - 1) // NUM_WORKERS


def _gate_part_body(x_ref, idxr_ref, w1_ref, b1_ref, w2_ref, b2_ref,
                    part_ref, gate_ref):
    i = pl.program_id(0)
    x = x_ref[...]
    h = jnp.maximum(
        jnp.dot(x.astype(jnp.bfloat16), w1_ref[...],
                preferred_element_type=jnp.float32) + b1_ref[...],
        0.0,
    )
    logit = jnp.dot(h.astype(jnp.bfloat16), w2_ref[...],
                    preferred_element_type=jnp.float32) + b2_ref[...]
    rows = i * BLK + lax.broadcasted_iota(jnp.int32, (BLK, 1), 0)
    gate_col = jnp.where(rows < N_NODES, jax.nn.sigmoid(logit), 0.0)
    gate_ref[...] = gate_col
    gated = jnp.where(rows < N_NODES, gate_col * x, 0.0)

    idxr = idxr_ref[...]
    base = idxr_ref[0, 0]
    span = idxr_ref[BLK - 1, 0] - base
    rel = idxr - base
    onehot = (lax.broadcasted_iota(jnp.int32, (BLK, W), 1) == rel)
    s = jnp.where(span < W, onehot.astype(jnp.bfloat16), 0.0)
    part_ref[...] = lax.dot_general(
        s, gated.astype(jnp.bfloat16), (((0,), (0,)), ((), ())),
        preferred_element_type=jnp.float32)


def _gate_and_partials(x, idx_blocks, W1, b1t, w2t, b2m):
    return pl.pallas_call(
        _gate_part_body,
        grid=(NBLK,),
        in_specs=[
            pl.BlockSpec((BLK, HIDDEN), lambda i: (i, 0)),
            pl.BlockSpec((BLK, 1), lambda i: (i, 0)),
            pl.BlockSpec((HIDDEN, HIDDEN), lambda i: (0, 0)),
            pl.BlockSpec((1, HIDDEN), lambda i: (0, 0)),
            pl.BlockSpec((HIDDEN, 1), lambda i: (0, 0)),
            pl.BlockSpec((1, 1), lambda i: (0, 0)),
        ],
        out_specs=[
            pl.BlockSpec((W, HIDDEN), lambda i: (i, 0)),
            pl.BlockSpec((BLK, 1), lambda i: (i, 0)),
        ],
        out_shape=[
            jax.ShapeDtypeStruct((NBLK * W, HIDDEN), jnp.float32),
            jax.ShapeDtypeStruct((N_PAD, 1), jnp.float32),
        ],
    )(x, idx_blocks, W1, b1t, w2t, b2m)


def _mult_rows(buf, gate_v, j):
    base = jnp.int32(j) * CHUNK if isinstance(j, int) else j * CHUNK

    def mgroup(g, carry):
        gvec = gate_v[pl.ds(base + g * 16, 16)]
        r0 = g * 16
        for t in range(16):
            g16 = lax.broadcast(gvec[t], (16,))
            for k in range(HIDDEN // 16):
                sl = pl.ds(k * 16, 16)
                buf[r0 + t, sl] = buf[r0 + t, sl] * g16
        return carry

    lax.fori_loop(0, CHUNK // 16, mgroup, 0)


def _seg_body(x_hbm, gate_hbm, idx_hbm, part_hbm, init_hbm, out_hbm,
              idx_v, gate_v, pbuf, xbuf, sidx, acc):
    c = lax.axis_index("c")
    s = lax.axis_index("s")
    w = c * NS + s
    pltpu.sync_copy(
        init_hbm.at[pl.ds(c * NUM_SEGMENTS + s * SEG_PER_SUB, SEG_PER_SUB)],
        acc.at[pl.ds(s * SEG_PER_SUB, SEG_PER_SUB)],
    )
    plsc.subcore_barrier()
    iota16 = lax.iota(jnp.int32, 16)

    for r in range(ROUNDS):
        b = r * NUM_WORKERS + w

        @pl.when(b < NBLK)
        def _():
            pltpu.sync_copy(idx_hbm.at[pl.ds(b * CPB, CPB)], idx_v)
            first = idx_v[0, pl.ds(0, 16)][0]
            last = idx_v[CPB - 1, pl.ds(CHUNK - 16, 16)][15]
            span = last - first

            @pl.when(span < W)
            def _():
                pltpu.sync_copy(part_hbm.at[pl.ds(b * W, W)], pbuf)
                f16 = lax.broadcast(first, (16,))
                sidx[pl.ds(0, 16)] = jnp.minimum(
                    f16 + iota16, NUM_SEGMENTS - 1)
                sidx[pl.ds(16, 16)] = jnp.minimum(
                    f16 + 16 + iota16, NUM_SEGMENTS - 1)
                pltpu.sync_copy(pbuf, acc.at[sidx], add=True)

            @pl.when(span >= W)
            def _():
                pltpu.sync_copy(
                    gate_hbm.at[pl.ds(b * BLK, BLK)], gate_v)
                for j in range(CPB):
                    g = b * CPB + j

                    @pl.when(g <= LAST_FULL_CHUNK)
                    def _():
                        pltpu.sync_copy(
                            x_hbm.at[pl.ds(g * CHUNK, CHUNK)], xbuf)
                        _mult_rows(xbuf, gate_v, j)
                        pltpu.sync_copy(
                            xbuf, acc.at[idx_v.at[j]], add=True)

                    @pl.when(g == TAIL_CHUNK)
                    def _():
                        zero16 = jnp.zeros((16,), jnp.float32)

                        def zrow(rr, carry):
                            for k in range(HIDDEN // 16):
                                xbuf[rr, pl.ds(k * 16, 16)] = zero16
                            return carry

                        lax.fori_loop(TAIL_ROWS, CHUNK, zrow, 0)
                        pltpu.sync_copy(
                            x_hbm.at[pl.ds(g * CHUNK, TAIL_ROWS)],
                            xbuf.at[pl.ds(0, TAIL_ROWS)])
                        _mult_rows(xbuf, gate_v, j)
                        pltpu.sync_copy(
                            xbuf, acc.at[idx_v.at[j]], add=True)

    plsc.subcore_barrier()
    pltpu.sync_copy(
        acc.at[pl.ds(s * SEG_PER_SUB, SEG_PER_SUB)],
        out_hbm.at[pl.ds(c * NUM_SEGMENTS + s * SEG_PER_SUB, SEG_PER_SUB)],
    )


def _seg_scatter(x, gate_flat, idx_chunks, partials, init):
    mesh = plsc.VectorSubcoreMesh(core_axis_name="c", subcore_axis_name="s")
    f = functools.partial(
        pl.kernel,
        mesh=mesh,
        out_type=jax.ShapeDtypeStruct((2 * NUM_SEGMENTS, HIDDEN), jnp.float32),
        scratch_types=[
            pltpu.VMEM((CPB, CHUNK), jnp.int32),
            pltpu.VMEM((BLK,), jnp.float32),
            pltpu.VMEM((W, HIDDEN), jnp.float32),
            pltpu.VMEM((CHUNK, HIDDEN), jnp.float32),
            pltpu.VMEM((W,), jnp.int32),
            pltpu.VMEM_SHARED((NUM_SEGMENTS, HIDDEN), jnp.float32),
        ],
    )(_seg_body)
    return f(x, gate_flat, idx_chunks, partials, init)


def kernel(node_embeddings, batch_idx, W1, b1, W2, b2):
    idx = batch_idx.astype(jnp.int32)
    idx_full = jnp.concatenate(
        [idx, jnp.broadcast_to(idx[-1:], (N_PAD - N_NODES,))])
    idx_blocks = idx_full.reshape(N_PAD, 1)
    idx_chunks = idx_full.reshape(N_CHUNKS, CHUNK)

    b1t = b1.reshape(1, HIDDEN)
    w1b = W1.astype(jnp.bfloat16)
    w2b = W2.reshape(HIDDEN, 1).astype(jnp.bfloat16)
    b2m = b2.reshape(1, 1)

    partials, gates = _gate_and_partials(
        node_embeddings, idx_blocks, w1b, b1t, w2b, b2m)
    init = jnp.zeros((2 * NUM_SEGMENTS, HIDDEN), jnp.float32)
    out2 = _seg_scatter(
        node_embeddings, gates.reshape(-1), idx_chunks, partials, init)
    return out2.reshape(2, NUM_SEGMENTS, HIDDEN).sum(axis=0)


# speedup vs baseline: 1.2594x; 1.2594x over previous
import functools

import jax
import jax.numpy as jnp
from jax import lax
from jax.experimental import pallas as pl
from jax.experimental.pallas import tpu as pltpu
from jax.experimental.pallas import tpu_sc as plsc

N_NODES = 100000
HIDDEN = 128
NUM_SEGMENTS = 1024

BLK = 1024
NBLK = (N_NODES + BLK - 1) // BLK
N_PAD = NBLK * BLK
W = 32

CHUNK = 128
CPB = BLK // CHUNK
N_CHUNKS = NBLK * CPB
LAST_FULL_CHUNK = N_NODES // CHUNK - 1
TAIL_CHUNK = 781
TAIL_ROWS = N_NODES - TAIL_CHUNK * CHUNK

NUM_WORKERS = 32
NS = 16
SEG_PER_SUB = NUM_SEGMENTS // NS
ROUNDS = (NBLK + NUM_WORKERS - 1) // NUM_WORKERS


DEPTH = 7
NSTEP = NBLK // DEPTH
LAST_BLK_ROWS = N_NODES - (NBLK - 1) * BLK


def _full_fetch(x_hbm, xbufs, sems, j, d):
    return pltpu.make_async_copy(
        x_hbm.at[pl.ds(j * BLK, BLK)], xbufs.at[d], sems.at[d])


def _tail_fetch(x_hbm, xbufs, sems, d):
    return pltpu.make_async_copy(
        x_hbm.at[pl.ds((NBLK - 1) * BLK, LAST_BLK_ROWS)],
        xbufs.at[d].at[pl.ds(0, LAST_BLK_ROWS)], sems.at[d])


def _gate_part_body(x_hbm, idxr_ref, w1_ref, b1_ref, w2_ref, b2_ref,
                    part_ref, gate_ref, xbufs, sems):
    k = pl.program_id(0)

    @pl.when(k == 0)
    def _():
        for d in range(DEPTH):
            _full_fetch(x_hbm, xbufs, sems, d, d).start()

    for d in range(DEPTH):
        j = k * DEPTH + d
        is_tail = d == DEPTH - 1

        if is_tail:
            @pl.when(k < NSTEP - 1)
            def _():
                _full_fetch(x_hbm, xbufs, sems, j, d).wait()

            @pl.when(k == NSTEP - 1)
            def _():
                _tail_fetch(x_hbm, xbufs, sems, d).wait()
        else:
            _full_fetch(x_hbm, xbufs, sems, j, d).wait()

        x = xbufs[d]
        h = jnp.maximum(
            jnp.dot(x, w1_ref[...], preferred_element_type=jnp.float32)
            + b1_ref[...],
            0.0,
        )
        logit = jnp.dot(h, w2_ref[...], preferred_element_type=jnp.float32) \
            + b2_ref[...]
        rows = j * BLK + lax.broadcasted_iota(jnp.int32, (BLK, 1), 0)
        gate_col = jnp.where(rows < N_NODES, jax.nn.sigmoid(logit), 0.0)
        gate_ref[pl.ds(d * BLK, BLK), :] = gate_col
        gated = jnp.where(rows < N_NODES, gate_col * x, 0.0)

        idxr = idxr_ref[pl.ds(d * BLK, BLK), :]
        base = idxr_ref[d * BLK, 0]
        span = idxr_ref[(d + 1) * BLK - 1, 0] - base
        rel = idxr - base
        onehot = (lax.broadcasted_iota(jnp.int32, (BLK, W), 1) == rel)
        s = jnp.where(span < W, onehot.astype(jnp.float32), 0.0)
        part_ref[pl.ds(d * W, W), :] = lax.dot_general(
            s, gated, (((0,), (0,)), ((), ())),
            preferred_element_type=jnp.float32)

        nj = j + DEPTH
        if is_tail:
            @pl.when(k == NSTEP - 2)
            def _():
                _tail_fetch(x_hbm, xbufs, sems, d).start()

            @pl.when(k < NSTEP - 2)
            def _():
                _full_fetch(x_hbm, xbufs, sems, nj, d).start()
        else:
            @pl.when(k < NSTEP - 1)
            def _():
                _full_fetch(x_hbm, xbufs, sems, nj, d).start()


def _gate_and_partials(x, idx_blocks, W1, b1t, w2t, b2m):
    return pl.pallas_call(
        _gate_part_body,
        grid=(NSTEP,),
        in_specs=[
            pl.BlockSpec(memory_space=pl.ANY),
            pl.BlockSpec((DEPTH * BLK, 1), lambda i: (i, 0)),
            pl.BlockSpec((HIDDEN, HIDDEN), lambda i: (0, 0)),
            pl.BlockSpec((1, HIDDEN), lambda i: (0, 0)),
            pl.BlockSpec((HIDDEN, 1), lambda i: (0, 0)),
            pl.BlockSpec((1, 1), lambda i: (0, 0)),
        ],
        out_specs=[
            pl.BlockSpec((DEPTH * W, HIDDEN), lambda i: (i, 0)),
            pl.BlockSpec((DEPTH * BLK, 1), lambda i: (i, 0)),
        ],
        out_shape=[
            jax.ShapeDtypeStruct((NBLK * W, HIDDEN), jnp.float32),
            jax.ShapeDtypeStruct((N_PAD, 1), jnp.float32),
        ],
        scratch_shapes=[
            pltpu.VMEM((DEPTH, BLK, HIDDEN), jnp.float32),
            pltpu.SemaphoreType.DMA((DEPTH,)),
        ],
    )(x, idx_blocks, W1, b1t, w2t, b2m)


def _mult_rows(buf, gate_v, j):
    base = jnp.int32(j) * CHUNK if isinstance(j, int) else j * CHUNK

    def mgroup(g, carry):
        gvec = gate_v[pl.ds(base + g * 16, 16)]
        r0 = g * 16
        for t in range(16):
            g16 = lax.broadcast(gvec[t], (16,))
            for k in range(HIDDEN // 16):
                sl = pl.ds(k * 16, 16)
                buf[r0 + t, sl] = buf[r0 + t, sl] * g16
        return carry

    lax.fori_loop(0, CHUNK // 16, mgroup, 0)


def _seg_body(x_hbm, gate_hbm, idx_hbm, part_hbm, init_hbm, out_hbm,
              idx_v, gate_v, pbuf, xbuf, sidx, acc):
    c = lax.axis_index("c")
    s = lax.axis_index("s")
    w = c * NS + s
    pltpu.sync_copy(
        init_hbm.at[pl.ds(c * NUM_SEGMENTS + s * SEG_PER_SUB, SEG_PER_SUB)],
        acc.at[pl.ds(s * SEG_PER_SUB, SEG_PER_SUB)],
    )
    plsc.subcore_barrier()
    iota16 = lax.iota(jnp.int32, 16)

    for r in range(ROUNDS):
        b = r * NUM_WORKERS + w

        @pl.when(b < NBLK)
        def _():
            pltpu.sync_copy(idx_hbm.at[pl.ds(b * CPB, CPB)], idx_v)
            first = idx_v[0, pl.ds(0, 16)][0]
            last = idx_v[CPB - 1, pl.ds(CHUNK - 16, 16)][15]
            span = last - first

            @pl.when(span < W)
            def _():
                pltpu.sync_copy(part_hbm.at[pl.ds(b * W, W)], pbuf)
                f16 = lax.broadcast(first, (16,))
                sidx[pl.ds(0, 16)] = jnp.minimum(
                    f16 + iota16, NUM_SEGMENTS - 1)
                sidx[pl.ds(16, 16)] = jnp.minimum(
                    f16 + 16 + iota16, NUM_SEGMENTS - 1)
                pltpu.sync_copy(pbuf, acc.at[sidx], add=True)

            @pl.when(span >= W)
            def _():
                pltpu.sync_copy(
                    gate_hbm.at[pl.ds(b * BLK, BLK)], gate_v)
                for j in range(CPB):
                    g = b * CPB + j

                    @pl.when(g <= LAST_FULL_CHUNK)
                    def _():
                        pltpu.sync_copy(
                            x_hbm.at[pl.ds(g * CHUNK, CHUNK)], xbuf)
                        _mult_rows(xbuf, gate_v, j)
                        pltpu.sync_copy(
                            xbuf, acc.at[idx_v.at[j]], add=True)

                    @pl.when(g == TAIL_CHUNK)
                    def _():
                        zero16 = jnp.zeros((16,), jnp.float32)

                        def zrow(rr, carry):
                            for k in range(HIDDEN // 16):
                                xbuf[rr, pl.ds(k * 16, 16)] = zero16
                            return carry

                        lax.fori_loop(TAIL_ROWS, CHUNK, zrow, 0)
                        pltpu.sync_copy(
                            x_hbm.at[pl.ds(g * CHUNK, TAIL_ROWS)],
                            xbuf.at[pl.ds(0, TAIL_ROWS)])
                        _mult_rows(xbuf, gate_v, j)
                        pltpu.sync_copy(
                            xbuf, acc.at[idx_v.at[j]], add=True)

    plsc.subcore_barrier()
    pltpu.sync_copy(
        acc.at[pl.ds(s * SEG_PER_SUB, SEG_PER_SUB)],
        out_hbm.at[pl.ds(c * NUM_SEGMENTS + s * SEG_PER_SUB, SEG_PER_SUB)],
    )


def _seg_scatter(x, gate_flat, idx_chunks, partials, init):
    mesh = plsc.VectorSubcoreMesh(core_axis_name="c", subcore_axis_name="s")
    f = functools.partial(
        pl.kernel,
        mesh=mesh,
        out_type=jax.ShapeDtypeStruct((2 * NUM_SEGMENTS, HIDDEN), jnp.float32),
        scratch_types=[
            pltpu.VMEM((CPB, CHUNK), jnp.int32),
            pltpu.VMEM((BLK,), jnp.float32),
            pltpu.VMEM((W, HIDDEN), jnp.float32),
            pltpu.VMEM((CHUNK, HIDDEN), jnp.float32),
            pltpu.VMEM((W,), jnp.int32),
            pltpu.VMEM_SHARED((NUM_SEGMENTS, HIDDEN), jnp.float32),
        ],
    )(_seg_body)
    return f(x, gate_flat, idx_chunks, partials, init)


def kernel(node_embeddings, batch_idx, W1, b1, W2, b2):
    idx = batch_idx.astype(jnp.int32)
    idx_full = jnp.concatenate(
        [idx, jnp.broadcast_to(idx[-1:], (N_PAD - N_NODES,))])
    idx_blocks = idx_full.reshape(N_PAD, 1)
    idx_chunks = idx_full.reshape(N_CHUNKS, CHUNK)

    b1t = b1.reshape(1, HIDDEN)
    w2c = W2.reshape(HIDDEN, 1)
    b2m = b2.reshape(1, 1)

    partials, gates = _gate_and_partials(
        node_embeddings, idx_blocks, W1, b1t, w2c, b2m)
    init = jnp.zeros((2 * NUM_SEGMENTS, HIDDEN), jnp.float32)
    out2 = _seg_scatter(
        node_embeddings, gates.reshape(-1), idx_chunks, partials, init)
    return out2.reshape(2, NUM_SEGMENTS, HIDDEN).sum(axis=0)


# speedup vs baseline: 1.3852x; 1.0999x over previous
import functools

import jax
import jax.numpy as jnp
from jax import lax
from jax.experimental import pallas as pl
from jax.experimental.pallas import tpu as pltpu
from jax.experimental.pallas import tpu_sc as plsc

N_NODES = 100000
HIDDEN = 128
NUM_SEGMENTS = 1024

NUM_WORKERS = 32
NS = 16
SEG_PER_SUB = NUM_SEGMENTS // NS

CHUNK = 128
CPS = 8
NSLICES = 4
SLICE_CHUNKS = NUM_WORKERS * CPS
SLICE_ROWS = SLICE_CHUNKS * CHUNK
N_PAD = NSLICES * SLICE_ROWS
N_CHUNKS = N_PAD // CHUNK
LAST_FULL_CHUNK = N_NODES // CHUNK - 1
TAIL_CHUNK = 781
TAIL_ROWS = N_NODES - TAIL_CHUNK * CHUNK

TC_BLOCK = 1024
TC_BLOCKS_PER_SLICE = SLICE_ROWS // TC_BLOCK
LAST_REAL_BLOCK = (N_NODES - 1) // TC_BLOCK
NBUF = 4


def _gate_body(x_ref, w1_ref, b1_ref, w2t_ref, b2_ref, out_ref, *, g0):
    i = pl.program_id(0)
    x = x_ref[...]
    h = jnp.maximum(
        jnp.dot(x, w1_ref[...], preferred_element_type=jnp.float32) + b1_ref[...],
        0.0,
    )
    logit = jnp.sum(h * w2t_ref[...], axis=1, keepdims=True) + b2_ref[...]
    gate = jax.nn.sigmoid(logit).reshape(TC_BLOCK // HIDDEN, HIDDEN)
    row0 = (g0 + i) * TC_BLOCK
    rows = (row0
            + HIDDEN * lax.broadcasted_iota(jnp.int32, gate.shape, 0)
            + lax.broadcasted_iota(jnp.int32, gate.shape, 1))
    out_ref[...] = jnp.where(rows < N_NODES, gate, 0.0)


def _gate_slice(p, n_blocks, x, W1, b1t, w2t, b2m):
    g0 = p * TC_BLOCKS_PER_SLICE
    return pl.pallas_call(
        functools.partial(_gate_body, g0=g0),
        grid=(n_blocks,),
        in_specs=[
            pl.BlockSpec((TC_BLOCK, HIDDEN),
                         lambda i: (jnp.minimum(g0 + i, LAST_REAL_BLOCK), 0)),
            pl.BlockSpec((HIDDEN, HIDDEN), lambda i: (0, 0)),
            pl.BlockSpec((1, HIDDEN), lambda i: (0, 0)),
            pl.BlockSpec((1, HIDDEN), lambda i: (0, 0)),
            pl.BlockSpec((1, 1), lambda i: (0, 0)),
        ],
        out_specs=pl.BlockSpec((TC_BLOCK // HIDDEN, HIDDEN), lambda i: (i, 0)),
        out_shape=jax.ShapeDtypeStruct((SLICE_CHUNKS, HIDDEN), jnp.float32),
    )(x, W1, b1t, w2t, b2m)


def _mult_rows(buf, gate_v, j):
    base = (jnp.int32(j) if isinstance(j, int) else j) * CHUNK

    def mgroup(g, carry):
        gvec = gate_v[pl.ds(base + g * 16, 16)]
        r0 = g * 16
        for t in range(16):
            g16 = lax.broadcast(gvec[t], (16,))
            for k in range(HIDDEN // 16):
                sl = pl.ds(k * 16, 16)
                buf[r0 + t, sl] = buf[r0 + t, sl] * g16
        return carry

    lax.fori_loop(0, CHUNK // 16, mgroup, 0)


def _seed_and_finish(init_hbm, out_hbm, acc, c, s):
    pltpu.sync_copy(
        init_hbm.at[pl.ds(c * NUM_SEGMENTS + s * SEG_PER_SUB, SEG_PER_SUB)],
        acc.at[pl.ds(s * SEG_PER_SUB, SEG_PER_SUB)],
    )
    plsc.subcore_barrier()

    def finish():
        plsc.subcore_barrier()
        pltpu.sync_copy(
            acc.at[pl.ds(s * SEG_PER_SUB, SEG_PER_SUB)],
            out_hbm.at[pl.ds(c * NUM_SEGMENTS + s * SEG_PER_SUB, SEG_PER_SUB)],
        )
    return finish


def _make_seg_body(p):
    def body(x_hbm, gate_hbm, idx_hbm, init_hbm, out_hbm,
             idx_v, gate_v, bufs0, bufs1, bufs2, bufs3, acc,
             d0, d1, d2, d3, t0, t1, t2, t3):
        c = lax.axis_index("c")
        s = lax.axis_index("s")
        finish = _seed_and_finish(init_hbm, out_hbm, acc, c, s)
        w = c * NS + s
        bufs = (bufs0, bufs1, bufs2, bufs3)
        dsem = (d0, d1, d2, d3)
        tsem = (t0, t1, t2, t3)

        if p < NSLICES - 1:
            base_chunk = p * SLICE_CHUNKS + w * CPS
            pltpu.sync_copy(idx_hbm.at[pl.ds(base_chunk, CPS)], idx_v)
            pltpu.sync_copy(
                gate_hbm.at[pl.ds(w * CPS * CHUNK, CPS * CHUNK)], gate_v)

            def start_dma(j):
                return pltpu.async_copy(
                    x_hbm.at[pl.ds((base_chunk + j) * CHUNK, CHUNK)],
                    bufs[j % NBUF], dsem[j % NBUF])

            dma = [None] * CPS
            scat = [None] * CPS
            for j in range(NBUF):
                dma[j] = start_dma(j)
            for j in range(CPS):
                dma[j].wait()
                _mult_rows(bufs[j % NBUF], gate_v, j)
                scat[j] = pltpu.async_copy(
                    bufs[j % NBUF], acc.at[idx_v.at[j]], tsem[j % NBUF],
                    add=True)
                if 1 <= j and j + 3 < CPS:
                    scat[j - 1].wait()
                    dma[j + 3] = start_dma(j + 3)
            for j in range(CPS - NBUF, CPS):
                scat[j].wait()
        else:
            g_chunk = p * SLICE_CHUNKS + w
            idx_block = p * SLICE_CHUNKS + 8 * (w // 8)
            pltpu.sync_copy(idx_hbm.at[pl.ds(idx_block, 8)], idx_v)
            pltpu.sync_copy(
                gate_hbm.at[pl.ds(8 * (w // 8) * CHUNK, 8 * CHUNK)], gate_v)
            jj = w % 8
            is_full = g_chunk <= LAST_FULL_CHUNK
            is_tail = g_chunk == TAIL_CHUNK
            buf = bufs[0]

            @pl.when(is_full)
            def _():
                pltpu.sync_copy(x_hbm.at[pl.ds(g_chunk * CHUNK, CHUNK)], buf)

            @pl.when(is_tail)
            def _():
                zero16 = jnp.zeros((16,), jnp.float32)

                def zrow(r, carry):
                    for k in range(HIDDEN // 16):
                        buf[r, pl.ds(k * 16, 16)] = zero16
                    return carry

                lax.fori_loop(TAIL_ROWS, CHUNK, zrow, 0)
                pltpu.sync_copy(
                    x_hbm.at[pl.ds(g_chunk * CHUNK, TAIL_ROWS)],
                    buf.at[pl.ds(0, TAIL_ROWS)])

            @pl.when(is_full | is_tail)
            def _():
                _mult_rows(buf, gate_v, jj)
                pltpu.sync_copy(buf, acc.at[idx_v.at[jj]], add=True)

        finish()
    return body


def _seg_sum_slice(p, x, gate_p, idx_all, init):
    mesh = plsc.VectorSubcoreMesh(core_axis_name="c", subcore_axis_name="s")
    f = functools.partial(
        pl.kernel,
        mesh=mesh,
        out_type=jax.ShapeDtypeStruct((2 * NUM_SEGMENTS, HIDDEN), jnp.float32),
        scratch_types=[
            pltpu.VMEM((CPS, CHUNK), jnp.int32),
            pltpu.VMEM((CPS * CHUNK,), jnp.float32),
            pltpu.VMEM((CHUNK, HIDDEN), jnp.float32),
            pltpu.VMEM((CHUNK, HIDDEN), jnp.float32),
            pltpu.VMEM((CHUNK, HIDDEN), jnp.float32),
            pltpu.VMEM((CHUNK, HIDDEN), jnp.float32),
            pltpu.VMEM_SHARED((NUM_SEGMENTS, HIDDEN), jnp.float32),
            pltpu.SemaphoreType.DMA,
            pltpu.SemaphoreType.DMA,
            pltpu.SemaphoreType.DMA,
            pltpu.SemaphoreType.DMA,
            pltpu.SemaphoreType.DMA,
            pltpu.SemaphoreType.DMA,
            pltpu.SemaphoreType.DMA,
            pltpu.SemaphoreType.DMA,
        ],
    )(_make_seg_body(p))
    return f(x, gate_p.reshape(-1), idx_all, init)


def kernel(node_embeddings, batch_idx, W1, b1, W2, b2):
    idx = batch_idx.astype(jnp.int32)
    idx_pad = jnp.concatenate(
        [idx, jnp.zeros((N_PAD - N_NODES,), jnp.int32)]
    ).reshape(N_CHUNKS, CHUNK)

    b1t = b1.reshape(1, HIDDEN)
    w2t = W2.reshape(HIDDEN, 1).T
    b2m = b2.reshape(1, 1)

    gates = [_gate_slice(p, TC_BLOCKS_PER_SLICE if p < NSLICES - 1 else 2,
                         node_embeddings, W1, b1t, w2t, b2m)
             for p in range(NSLICES)]
    partial = jnp.zeros((2 * NUM_SEGMENTS, HIDDEN), jnp.float32)
    for p in range(NSLICES):
        partial = _seg_sum_slice(p, node_embeddings, gates[p], idx_pad, partial)
    return partial.reshape(2, NUM_SEGMENTS, HIDDEN).sum(axis=0)
